# baseline (speedup 1.0000x reference)
import functools

import jax
import jax.numpy as jnp
from jax import lax
from jax.experimental import pallas as pl
from jax.experimental.pallas import tpu as pltpu
from jax.experimental.pallas import tpu_sc as plsc

NCORES = 2
NSUB = 16
NWORK = NCORES * NSUB
LANES = 16
SEGS = 6
TLEN = 64
NODES = SEGS * (TLEN - 1)
TPAD = 384
CH = 16384
UNROLL = 4


def _sc_lut(x, tab_merged, consts):
    n = x.shape[0]
    per_w = n // NWORK
    n_chunks = per_w // CH

    mesh = plsc.VectorSubcoreMesh(
        core_axis_name="c", subcore_axis_name="s",
        num_cores=NCORES, num_subcores=NSUB)

    @functools.partial(
        pl.kernel,
        out_type=jax.ShapeDtypeStruct((n,), jnp.float32),
        mesh=mesh,
        scratch_types=[
            pltpu.VMEM((TPAD,), jnp.float32),
            pltpu.VMEM((3, LANES), jnp.float32),
            pltpu.VMEM((2, CH), jnp.float32),
            pltpu.VMEM((2, CH), jnp.float32),
            pltpu.SemaphoreType.DMA,
            pltpu.SemaphoreType.DMA,
        ],
        compiler_params=pltpu.CompilerParams(needs_layout_passes=False),
    )
    def k(x_hbm, tab_hbm, consts_hbm, out_hbm,
          tab_v, c_v, in_v, out_v, in_sem, out_sem):
        wid = lax.axis_index("s") * NCORES + lax.axis_index("c")
        base = wid * per_w
        pltpu.sync_copy(tab_hbm, tab_v)
        pltpu.sync_copy(consts_hbm, c_v)
        lo0 = c_v[0]
        hi0 = c_v[1]
        inv = c_v[2]

        def in_dma(g):
            return pltpu.async_copy(
                x_hbm.at[pl.ds(base + g * CH, CH)], in_v.at[g % 2], in_sem)

        def out_dma(g):
            return pltpu.async_copy(
                out_v.at[g % 2], out_hbm.at[pl.ds(base + g * CH, CH)], out_sem)

        in_descs = {0: in_dma(0)}
        out_descs = {}
        for g in range(n_chunks):
            if g + 1 < n_chunks:
                in_descs[g + 1] = in_dma(g + 1)
            in_descs.pop(g).wait()
            if g >= 2:
                out_descs.pop(g - 2).wait()
            b = g % 2

            def vec_body(i, _, b=b):
                off = i * (LANES * UNROLL)
                for u in range(UNROLL):
                    o = off + u * LANES
                    xv = in_v[b, pl.ds(o, LANES)]
                    xc = jnp.minimum(jnp.maximum(xv, lo0), hi0)
                    t = (xc - lo0) * inv
                    ti = jnp.minimum(t.astype(jnp.int32), NODES - 1)
                    frac = t - ti.astype(jnp.float32)
                    y0 = plsc.load_gather(tab_v, [ti])
                    y1 = plsc.load_gather(tab_v, [ti + 1])
                    out_v[b, pl.ds(o, LANES)] = y0 * (1.0 - frac) + y1 * frac
                return 0

            lax.fori_loop(0, CH // (LANES * UNROLL), vec_body, 0)
            out_descs[g] = out_dma(g)
        out_descs.pop(n_chunks - 2).wait()
        out_descs.pop(n_chunks - 1).wait()

    return k(x, tab_merged, consts)


def kernel(x, table, dividing_points):
    tab_merged = jnp.concatenate([
        table[:, : TLEN - 1].reshape(-1),
        table[SEGS - 1:, TLEN - 1],
        jnp.zeros((TPAD - NODES - 1,), jnp.float32),
    ])
    lo0 = dividing_points[0]
    hi0 = dividing_points[-1]
    inv = NODES / (hi0 - lo0)
    consts = jnp.stack([
        jnp.full((LANES,), lo0, jnp.float32),
        jnp.full((LANES,), hi0, jnp.float32),
        jnp.full((LANES,), inv, jnp.float32),
    ])
    return _sc_lut(x, tab_merged, consts)


# speedup vs baseline: 1.5367x; 1.5367x over previous
import functools

import jax
import jax.numpy as jnp
from jax import lax
from jax.experimental import pallas as pl
from jax.experimental.pallas import tpu as pltpu
from jax.experimental.pallas import tpu_sc as plsc

NCORES = 2
NSUB = 16
NWORK = NCORES * NSUB
LANES = 16
SEGS = 6
TLEN = 64
NODES = SEGS * (TLEN - 1)
TPAD = 384
CH = 16384


def _sc_lut(x, tab_merged, consts):
    n = x.shape[0]
    per_w = n // NWORK
    n_chunks = per_w // CH

    mesh = plsc.VectorSubcoreMesh(
        core_axis_name="c", subcore_axis_name="s",
        num_cores=NCORES, num_subcores=NSUB)

    @functools.partial(
        pl.kernel,
        out_type=jax.ShapeDtypeStruct((n,), jnp.float32),
        mesh=mesh,
        scratch_types=[
            pltpu.VMEM((TPAD,), jnp.float32),
            pltpu.VMEM((3, LANES), jnp.float32),
            pltpu.VMEM((CH,), jnp.float32),
            pltpu.VMEM((CH,), jnp.float32),
        ],
        compiler_params=pltpu.CompilerParams(needs_layout_passes=False),
    )
    def k(x_hbm, tab_hbm, consts_hbm, out_hbm, tab_v, c_v, in_v, out_v):
        wid = lax.axis_index("s") * NCORES + lax.axis_index("c")
        base = wid * per_w
        pltpu.sync_copy(tab_hbm, tab_v)
        pltpu.sync_copy(consts_hbm, c_v)
        lo0 = c_v[0]
        hi0 = c_v[1]
        inv = c_v[2]

        def chunk_body(g, _):
            start = base + g * CH
            pltpu.sync_copy(x_hbm.at[pl.ds(start, CH)], in_v)

            def vec_body(i, _):
                o = i * LANES
                xv = in_v[pl.ds(o, LANES)]
                xc = jnp.minimum(jnp.maximum(xv, lo0), hi0)
                t = (xc - lo0) * inv
                ti = jnp.minimum(t.astype(jnp.int32), NODES - 1)
                frac = t - ti.astype(jnp.float32)
                y0 = plsc.load_gather(tab_v, [ti])
                y1 = plsc.load_gather(tab_v, [ti + 1])
                out_v[pl.ds(o, LANES)] = y0 * (1.0 - frac) + y1 * frac
                return 0

            lax.fori_loop(0, CH // LANES, vec_body, 0)
            pltpu.sync_copy(out_v, out_hbm.at[pl.ds(start, CH)])
            return 0

        lax.fori_loop(0, n_chunks, chunk_body, 0)

    return k(x, tab_merged, consts)


def kernel(x, table, dividing_points):
    tab_merged = jnp.concatenate([
        table[:, : TLEN - 1].reshape(-1),
        table[SEGS - 1:, TLEN - 1],
        jnp.zeros((TPAD - NODES - 1,), jnp.float32),
    ])
    lo0 = dividing_points[0]
    hi0 = dividing_points[-1]
    inv = NODES / (hi0 - lo0)
    consts = jnp.stack([
        jnp.full((LANES,), lo0, jnp.float32),
        jnp.full((LANES,), hi0, jnp.float32),
        jnp.full((LANES,), inv, jnp.float32),
    ])
    return _sc_lut(x, tab_merged, consts)


# speedup vs baseline: 2.4401x; 1.5879x over previous
import functools

import jax
import jax.numpy as jnp
from jax import lax
from jax.experimental import pallas as pl
from jax.experimental.pallas import tpu as pltpu
from jax.experimental.pallas import tpu_sc as plsc

NCORES = 2
NSUB = 16
NWORK = NCORES * NSUB
LANES = 16
SEGS = 6
TLEN = 64
NODES = SEGS * (TLEN - 1)
TPAD = 384
CH = 16384


def _sc_lut(x, tab_merged, consts):
    n = x.shape[0]
    per_w = n // NWORK
    n_chunks = per_w // CH

    mesh = plsc.VectorSubcoreMesh(
        core_axis_name="c", subcore_axis_name="s",
        num_cores=NCORES, num_subcores=NSUB)

    @functools.partial(
        pl.kernel,
        out_type=jax.ShapeDtypeStruct((n,), jnp.float32),
        mesh=mesh,
        scratch_types=[
            pltpu.VMEM((TPAD,), jnp.float32),
            pltpu.VMEM((3, LANES), jnp.float32),
            pltpu.VMEM((CH,), jnp.float32),
            pltpu.VMEM((CH,), jnp.float32),
        ],
        compiler_params=pltpu.CompilerParams(needs_layout_passes=False),
    )
    def k(x_hbm, tab_hbm, consts_hbm, out_hbm, tab_v, c_v, in_v, out_v):
        wid = lax.axis_index("s") * NCORES + lax.axis_index("c")
        base = wid * per_w
        pltpu.sync_copy(tab_hbm, tab_v)
        pltpu.sync_copy(consts_hbm, c_v)
        lo0 = c_v[0]
        hi0 = c_v[1]
        inv = c_v[2]

        def chunk_body(g, _):
            start = base + g * CH
            pltpu.sync_copy(x_hbm.at[pl.ds(start, CH)], in_v)

            @plsc.parallel_loop(0, CH, step=LANES, unroll=8)
            def vec_body(o):
                xv = in_v[pl.ds(o, LANES)]
                xc = jnp.minimum(jnp.maximum(xv, lo0), hi0)
                t = (xc - lo0) * inv
                ti = jnp.minimum(t.astype(jnp.int32), NODES - 1)
                frac = t - ti.astype(jnp.float32)
                y0 = plsc.load_gather(tab_v, [ti])
                y1 = plsc.load_gather(tab_v, [ti + 1])
                out_v[pl.ds(o, LANES)] = y0 * (1.0 - frac) + y1 * frac
            pltpu.sync_copy(out_v, out_hbm.at[pl.ds(start, CH)])
            return 0

        lax.fori_loop(0, n_chunks, chunk_body, 0)

    return k(x, tab_merged, consts)


def kernel(x, table, dividing_points):
    tab_merged = jnp.concatenate([
        table[:, : TLEN - 1].reshape(-1),
        table[SEGS - 1:, TLEN - 1],
        jnp.zeros((TPAD - NODES - 1,), jnp.float32),
    ])
    lo0 = dividing_points[0]
    hi0 = dividing_points[-1]
    inv = NODES / (hi0 - lo0)
    consts = jnp.stack([
        jnp.full((LANES,), lo0, jnp.float32),
        jnp.full((LANES,), hi0, jnp.float32),
        jnp.full((LANES,), inv, jnp.float32),
    ])
    return _sc_lut(x, tab_merged, consts)


# speedup vs baseline: 3.3149x; 1.3585x over previous
import functools

import jax
import jax.numpy as jnp
from jax import lax
from jax.experimental import pallas as pl
from jax.experimental.pallas import tpu as pltpu
from jax.experimental.pallas import tpu_sc as plsc

NCORES = 2
NSUB = 16
NWORK = NCORES * NSUB
LANES = 16
SEGS = 6
TLEN = 64
NODES = SEGS * (TLEN - 1)
TPAD = 384
CH = 16384


def _sc_lut(x, tab_merged, consts):
    n = x.shape[0]
    per_w = n // NWORK
    n_chunks = per_w // CH

    mesh = plsc.VectorSubcoreMesh(
        core_axis_name="c", subcore_axis_name="s",
        num_cores=NCORES, num_subcores=NSUB)

    @functools.partial(
        pl.kernel,
        out_type=jax.ShapeDtypeStruct((n,), jnp.float32),
        mesh=mesh,
        scratch_types=[
            pltpu.VMEM((TPAD,), jnp.float32),
            pltpu.VMEM((3, LANES), jnp.float32),
            pltpu.VMEM((2 * CH,), jnp.float32),
            pltpu.VMEM((2 * CH,), jnp.float32),
            pltpu.SemaphoreType.DMA,
            pltpu.SemaphoreType.DMA,
        ],
        compiler_params=pltpu.CompilerParams(needs_layout_passes=False),
    )
    def k(x_hbm, tab_hbm, consts_hbm, out_hbm,
          tab_v, c_v, in_v, out_v, in_sem, out_sem):
        wid = lax.axis_index("s") * NCORES + lax.axis_index("c")
        base = wid * per_w
        pltpu.sync_copy(tab_hbm, tab_v)
        pltpu.sync_copy(consts_hbm, c_v)
        lo0 = c_v[0]
        hi0 = c_v[1]
        inv = c_v[2]

        def in_copy(g, boff):
            return pltpu.make_async_copy(
                x_hbm.at[pl.ds(base + g * CH, CH)],
                in_v.at[pl.ds(boff, CH)], in_sem)

        def out_copy(g, boff):
            return pltpu.make_async_copy(
                out_v.at[pl.ds(boff, CH)],
                out_hbm.at[pl.ds(base + g * CH, CH)], out_sem)

        def compute(boff):
            @plsc.parallel_loop(0, CH, step=LANES, unroll=8)
            def vec_body(o):
                xv = in_v[pl.ds(boff + o, LANES)]
                xc = jnp.minimum(jnp.maximum(xv, lo0), hi0)
                t = (xc - lo0) * inv
                ti = jnp.minimum(t.astype(jnp.int32), NODES - 1)
                frac = t - ti.astype(jnp.float32)
                y0 = plsc.load_gather(tab_v, [ti])
                y1 = plsc.load_gather(tab_v, [ti + 1])
                out_v[pl.ds(boff + o, LANES)] = y0 * (1.0 - frac) + y1 * frac

        in_copy(0, 0).start()
        in_copy(1, CH).start()
        for g in (0, 1):
            boff = g * CH
            in_copy(g, boff).wait()
            compute(boff)
            out_copy(g, boff).start()
            in_copy(g + 2, boff).start()

        def steady(g, _):
            boff = (g % 2) * CH
            in_copy(g, boff).wait()
            out_copy(g - 2, boff).wait()
            compute(boff)
            out_copy(g, boff).start()
            in_copy(g + 2, boff).start()
            return 0

        lax.fori_loop(2, n_chunks - 2, steady, 0)
        for g in (n_chunks - 2, n_chunks - 1):
            boff = (g % 2) * CH
            in_copy(g, boff).wait()
            out_copy(g - 2, boff).wait()
            compute(boff)
            out_copy(g, boff).start()
        out_copy(n_chunks - 2, 0).wait()
        out_copy(n_chunks - 1, CH).wait()

    return k(x, tab_merged, consts)


def kernel(x, table, dividing_points):
    tab_merged = jnp.concatenate([
        table[:, : TLEN - 1].reshape(-1),
        table[SEGS - 1:, TLEN - 1],
        jnp.zeros((TPAD - NODES - 1,), jnp.float32),
    ])
    lo0 = dividing_points[0]
    hi0 = dividing_points[-1]
    inv = NODES / (hi0 - lo0)
    consts = jnp.stack([
        jnp.full((LANES,), lo0, jnp.float32),
        jnp.full((LANES,), hi0, jnp.float32),
        jnp.full((LANES,), inv, jnp.float32),
    ])
    return _sc_lut(x, tab_merged, consts)


# speedup vs baseline: 4.0048x; 1.2081x over previous
import functools

import jax
import jax.numpy as jnp
from jax import lax
from jax.experimental import pallas as pl
from jax.experimental.pallas import tpu as pltpu
from jax.experimental.pallas import tpu_sc as plsc

NCORES = 2
NSUB = 16
NWORK = NCORES * NSUB
LANES = 16
SEGS = 6
TLEN = 64
NODES = SEGS * (TLEN - 1)
TPAD = 400
TMAX = 377.99997
CH = 16384
UNROLL = 8


def _sc_lut(x, tab_merged, consts):
    n = x.shape[0]
    per_w = n // NWORK
    n_chunks = per_w // CH

    mesh = plsc.VectorSubcoreMesh(
        core_axis_name="c", subcore_axis_name="s",
        num_cores=NCORES, num_subcores=NSUB)

    @functools.partial(
        pl.kernel,
        out_type=jax.ShapeDtypeStruct((n,), jnp.float32),
        mesh=mesh,
        scratch_types=[
            pltpu.VMEM((TPAD,), jnp.float32),
            pltpu.VMEM((TPAD - LANES,), jnp.float32),
            pltpu.VMEM((2, LANES), jnp.float32),
            pltpu.VMEM((2 * CH,), jnp.float32),
            pltpu.VMEM((2 * CH,), jnp.float32),
            pltpu.SemaphoreType.DMA,
            pltpu.SemaphoreType.DMA,
        ],
        compiler_params=pltpu.CompilerParams(needs_layout_passes=False),
    )
    def k(x_hbm, tab_hbm, consts_hbm, out_hbm,
          tab_v, dy_v, c_v, in_v, out_v, in_sem, out_sem):
        wid = lax.axis_index("s") * NCORES + lax.axis_index("c")
        base = wid * per_w
        pltpu.sync_copy(tab_hbm, tab_v)
        pltpu.sync_copy(consts_hbm, c_v)
        inv = c_v[0]
        off = c_v[1]

        @plsc.parallel_loop(0, TPAD - LANES, step=LANES)
        def mk_dy(o):
            dy_v[pl.ds(o, LANES)] = (
                tab_v[pl.ds(o + 1, LANES)] - tab_v[pl.ds(o, LANES)])

        def in_copy(g, boff):
            return pltpu.make_async_copy(
                x_hbm.at[pl.ds(base + g * CH, CH)],
                in_v.at[pl.ds(boff, CH)], in_sem)

        def out_copy(g, boff):
            return pltpu.make_async_copy(
                out_v.at[pl.ds(boff, CH)],
                out_hbm.at[pl.ds(base + g * CH, CH)], out_sem)

        def compute(boff):
            @plsc.parallel_loop(0, CH, step=LANES, unroll=UNROLL)
            def vec_body(o):
                xv = in_v[pl.ds(boff + o, LANES)]
                t = jnp.minimum(jnp.maximum(xv * inv + off, 0.0), TMAX)
                ti = t.astype(jnp.int32)
                frac = t - ti.astype(jnp.float32)
                y0 = plsc.load_gather(tab_v, [ti])
                dy = plsc.load_gather(dy_v, [ti])
                out_v[pl.ds(boff + o, LANES)] = y0 + dy * frac

        in_copy(0, 0).start()
        in_copy(1, CH).start()
        for g in (0, 1):
            boff = g * CH
            in_copy(g, boff).wait()
            compute(boff)
            out_copy(g, boff).start()
            in_copy(g + 2, boff).start()

        def steady(g, _):
            boff = (g % 2) * CH
            in_copy(g, boff).wait()
            out_copy(g - 2, boff).wait()
            compute(boff)
            out_copy(g, boff).start()
            in_copy(g + 2, boff).start()
            return 0

        lax.fori_loop(2, n_chunks - 2, steady, 0)
        for g in (n_chunks - 2, n_chunks - 1):
            boff = (g % 2) * CH
            in_copy(g, boff).wait()
            out_copy(g - 2, boff).wait()
            compute(boff)
            out_copy(g, boff).start()
        out_copy(n_chunks - 2, 0).wait()
        out_copy(n_chunks - 1, CH).wait()

    return k(x, tab_merged, consts)


def kernel(x, table, dividing_points):
    tab_merged = jnp.concatenate([
        table[:, : TLEN - 1].reshape(-1),
        table[SEGS - 1:, TLEN - 1],
        jnp.zeros((TPAD - NODES - 1,), jnp.float32),
    ])
    lo0 = dividing_points[0]
    hi0 = dividing_points[-1]
    inv = NODES / (hi0 - lo0)
    consts = jnp.stack([
        jnp.full((LANES,), inv, jnp.float32),
        jnp.full((LANES,), -lo0 * inv, jnp.float32),
    ])
    return _sc_lut(x, tab_merged, consts)


# speedup vs baseline: 4.0218x; 1.0043x over previous
import functools

import jax
import jax.numpy as jnp
from jax import lax
from jax.experimental import pallas as pl
from jax.experimental.pallas import tpu as pltpu
from jax.experimental.pallas import tpu_sc as plsc

NCORES = 2
NSUB = 16
NWORK = NCORES * NSUB
LANES = 16
SEGS = 6
TLEN = 64
NODES = SEGS * (TLEN - 1)
TPAD = 400
TMAX = 377.99997
CH = 16384
UNROLL = 16


def _sc_lut(x, tab_merged, consts):
    n = x.shape[0]
    per_w = n // NWORK
    n_chunks = per_w // CH

    mesh = plsc.VectorSubcoreMesh(
        core_axis_name="c", subcore_axis_name="s",
        num_cores=NCORES, num_subcores=NSUB)

    @functools.partial(
        pl.kernel,
        out_type=jax.ShapeDtypeStruct((n,), jnp.float32),
        mesh=mesh,
        scratch_types=[
            pltpu.VMEM((TPAD,), jnp.float32),
            pltpu.VMEM((TPAD - LANES,), jnp.float32),
            pltpu.VMEM((2, LANES), jnp.float32),
            pltpu.VMEM((2 * CH,), jnp.float32),
            pltpu.VMEM((2 * CH,), jnp.float32),
            pltpu.SemaphoreType.DMA,
            pltpu.SemaphoreType.DMA,
        ],
        compiler_params=pltpu.CompilerParams(needs_layout_passes=False),
    )
    def k(x_hbm, tab_hbm, consts_hbm, out_hbm,
          tab_v, dy_v, c_v, in_v, out_v, in_sem, out_sem):
        wid = lax.axis_index("s") * NCORES + lax.axis_index("c")
        base = wid * per_w
        pltpu.sync_copy(tab_hbm, tab_v)
        pltpu.sync_copy(consts_hbm, c_v)
        inv = c_v[0]
        off = c_v[1]

        @plsc.parallel_loop(0, TPAD - LANES, step=LANES)
        def mk_dy(o):
            dy_v[pl.ds(o, LANES)] = (
                tab_v[pl.ds(o + 1, LANES)] - tab_v[pl.ds(o, LANES)])

        def in_copy(g, boff):
            return pltpu.make_async_copy(
                x_hbm.at[pl.ds(base + g * CH, CH)],
                in_v.at[pl.ds(boff, CH)], in_sem)

        def out_copy(g, boff):
            return pltpu.make_async_copy(
                out_v.at[pl.ds(boff, CH)],
                out_hbm.at[pl.ds(base + g * CH, CH)], out_sem)

        def compute(boff):
            @plsc.parallel_loop(0, CH, step=LANES, unroll=UNROLL)
            def vec_body(o):
                xv = in_v[pl.ds(boff + o, LANES)]
                t = jnp.minimum(jnp.maximum(xv * inv + off, 0.0), TMAX)
                ti = t.astype(jnp.int32)
                frac = t - ti.astype(jnp.float32)
                y0 = plsc.load_gather(tab_v, [ti])
                dy = plsc.load_gather(dy_v, [ti])
                out_v[pl.ds(boff + o, LANES)] = y0 + dy * frac

        in_copy(0, 0).start()
        in_copy(1, CH).start()
        for g in (0, 1):
            boff = g * CH
            in_copy(g, boff).wait()
            compute(boff)
            out_copy(g, boff).start()
            in_copy(g + 2, boff).start()

        def steady(g, _):
            boff = (g % 2) * CH
            in_copy(g, boff).wait()
            out_copy(g - 2, boff).wait()
            compute(boff)
            out_copy(g, boff).start()
            in_copy(g + 2, boff).start()
            return 0

        lax.fori_loop(2, n_chunks - 2, steady, 0)
        for g in (n_chunks - 2, n_chunks - 1):
            boff = (g % 2) * CH
            in_copy(g, boff).wait()
            out_copy(g - 2, boff).wait()
            compute(boff)
            out_copy(g, boff).start()
        out_copy(n_chunks - 2, 0).wait()
        out_copy(n_chunks - 1, CH).wait()

    return k(x, tab_merged, consts)


def kernel(x, table, dividing_points):
    tab_merged = jnp.concatenate([
        table[:, : TLEN - 1].reshape(-1),
        table[SEGS - 1:, TLEN - 1],
        jnp.zeros((TPAD - NODES - 1,), jnp.float32),
    ])
    lo0 = dividing_points[0]
    hi0 = dividing_points[-1]
    inv = NODES / (hi0 - lo0)
    consts = jnp.stack([
        jnp.full((LANES,), inv, jnp.float32),
        jnp.full((LANES,), -lo0 * inv, jnp.float32),
    ])
    return _sc_lut(x, tab_merged, consts)
